# Initial kernel scaffold; baseline (speedup 1.0000x reference)
import jax
import jax.numpy as jnp
from jax.experimental import pallas as pl


def kernel(in_feat, edge_index, W_self, W_neigh, b):
    raise NotImplementedError("write your pallas kernel here")


# speedup vs baseline: 10.7153x; 10.7153x over previous
import jax
import jax.numpy as jnp
from jax import lax
from jax.experimental import pallas as pl
from jax.experimental.pallas import tpu as pltpu
from jax.experimental.pallas import tpu_sc as plsc

_N = 10000
_E = 160000
_D = 256
_HALF = 128
_TILES = 16
_EPT = _E // _TILES
_K = 125
_NB = _EPT // _K
_NPAD = 10240
_RPT = _NPAD // _TILES
_RCH = 80
_DPT = _NPAD // _TILES


def _sc_aggregate(x_lo, x_hi, src3, dst3):
    mesh = plsc.VectorSubcoreMesh(core_axis_name="c", subcore_axis_name="s")

    def body(x0, x1, s4, d4, agg0, agg1, deg,
             agg_sp, deg_sp, rows0, rows1, si0, si1, di0, di1, ones_v, dbuf,
             sem0, sem1, isem0, isem1):
        c = lax.axis_index("c")
        s = lax.axis_index("s")
        ones16 = jnp.full((16,), 1.0, jnp.float32)

        def run(xc, aggc):
            for i in range(8):
                ones_v[pl.ds(i * 16, 16)] = ones16
            for i in range(_DPT // 16):
                dbuf[pl.ds(i * 16, 16)] = ones16
            stage = rows0.at[pl.ds(0, _RCH)]
            for j in range(_RPT // _RCH):
                base = s * _RPT + j * _RCH
                pltpu.sync_copy(xc.at[pl.ds(base, _RCH)], stage)
                pltpu.sync_copy(stage, agg_sp.at[pl.ds(base, _RCH)])
            pltpu.sync_copy(dbuf, deg_sp.at[pl.ds(s * _DPT, _DPT)])
            plsc.subcore_barrier()

            def fetch_idx(jj, si, di, isem):
                pltpu.async_copy(s4.at[s, jj, 0], si, isem)
                pltpu.async_copy(d4.at[s, jj, 0], di, isem)

            def wait_idx(jj, si, di, isem):
                pltpu.make_async_copy(s4.at[s, jj, 0], si, isem).wait()
                pltpu.make_async_copy(d4.at[s, jj, 0], di, isem).wait()

            pltpu.sync_copy(s4.at[s, 0, 0], si0)
            pltpu.sync_copy(d4.at[s, 0, 0], di0)
            pltpu.async_copy(xc.at[si0], rows0, sem0)
            fetch_idx(1, si1, di1, isem1)

            def step(h, carry):
                j0 = 2 * h
                j1 = j0 + 1
                wait_idx(j1, si1, di1, isem1)
                pltpu.async_copy(xc.at[si1], rows1, sem1)
                pltpu.make_async_copy(xc.at[si0], rows0, sem0).wait()
                pltpu.sync_copy(rows0, agg_sp.at[di0], add=True)
                pltpu.sync_copy(ones_v.at[pl.ds(0, _K)],
                                deg_sp.at[di0], add=True)

                @pl.when(h + 1 < _NB // 2)
                def _():
                    fetch_idx(j0 + 2, si0, di0, isem0)

                pltpu.make_async_copy(xc.at[si1], rows1, sem1).wait()
                pltpu.sync_copy(rows1, agg_sp.at[di1], add=True)
                pltpu.sync_copy(ones_v.at[pl.ds(0, _K)],
                                deg_sp.at[di1], add=True)

                @pl.when(h + 1 < _NB // 2)
                def _():
                    fetch_idx(j1 + 2, si1, di1, isem1)
                    wait_idx(j0 + 2, si0, di0, isem0)
                    pltpu.async_copy(xc.at[si0], rows0, sem0)

                return carry

            lax.fori_loop(0, _NB // 2, step, 0)
            plsc.subcore_barrier()

            for j in range(_RPT // _RCH):
                base = s * _RPT + j * _RCH
                pltpu.sync_copy(agg_sp.at[pl.ds(base, _RCH)], stage)
                pltpu.sync_copy(stage, aggc.at[pl.ds(base, _RCH)])

        @pl.when(c == 0)
        def _():
            run(x0, agg0)

        @pl.when(c == 1)
        def _():
            run(x1, agg1)

        @pl.when(c == 0)
        def _():
            pltpu.sync_copy(deg_sp.at[pl.ds(s * _DPT, _DPT)], dbuf)
            pltpu.sync_copy(dbuf, deg.at[pl.ds(s * _DPT, _DPT)])

    f = pl.kernel(
        body,
        out_type=[
            jax.ShapeDtypeStruct((_NPAD, _HALF), jnp.float32),
            jax.ShapeDtypeStruct((_NPAD, _HALF), jnp.float32),
            jax.ShapeDtypeStruct((_NPAD,), jnp.float32),
        ],
        mesh=mesh,
        scratch_types=[
            pltpu.VMEM_SHARED((_NPAD, _HALF), jnp.float32),
            pltpu.VMEM_SHARED((_NPAD,), jnp.float32),
            pltpu.VMEM((_K, _HALF), jnp.float32),
            pltpu.VMEM((_K, _HALF), jnp.float32),
            pltpu.VMEM((_K,), jnp.int32),
            pltpu.VMEM((_K,), jnp.int32),
            pltpu.VMEM((_K,), jnp.int32),
            pltpu.VMEM((_K,), jnp.int32),
            pltpu.VMEM((128,), jnp.float32),
            pltpu.VMEM((_DPT,), jnp.float32),
            pltpu.SemaphoreType.DMA,
            pltpu.SemaphoreType.DMA,
            pltpu.SemaphoreType.DMA,
            pltpu.SemaphoreType.DMA,
        ],
    )
    return f(x_lo, x_hi, src3, dst3)


_BM = 2000


def _tc_body(x_ref, a0_ref, a1_ref, deg_ref, ws_ref, wn0_ref, wn1_ref,
             b_ref, o_ref):
    r = 1.0 / deg_ref[...]
    acc = jnp.dot(x_ref[...], ws_ref[...], preferred_element_type=jnp.float32)
    acc += jnp.dot(a0_ref[...] * r, wn0_ref[...],
                   preferred_element_type=jnp.float32)
    acc += jnp.dot(a1_ref[...] * r, wn1_ref[...],
                   preferred_element_type=jnp.float32)
    o_ref[...] = acc + b_ref[...]


def _tc_combine(x, a0, a1, deg, w_self, w_neigh, b):
    wn0 = w_neigh[:_HALF]
    wn1 = w_neigh[_HALF:]
    deg2 = deg[:_N].reshape(_N, 1)
    b2 = b.reshape(1, _D)
    return pl.pallas_call(
        _tc_body,
        grid=(_N // _BM,),
        in_specs=[
            pl.BlockSpec((_BM, _D), lambda i: (i, 0)),
            pl.BlockSpec((_BM, _HALF), lambda i: (i, 0)),
            pl.BlockSpec((_BM, _HALF), lambda i: (i, 0)),
            pl.BlockSpec((_BM, 1), lambda i: (i, 0)),
            pl.BlockSpec((_D, _D), lambda i: (0, 0)),
            pl.BlockSpec((_HALF, _D), lambda i: (0, 0)),
            pl.BlockSpec((_HALF, _D), lambda i: (0, 0)),
            pl.BlockSpec((1, _D), lambda i: (0, 0)),
        ],
        out_specs=pl.BlockSpec((_BM, _D), lambda i: (i, 0)),
        out_shape=jax.ShapeDtypeStruct((_N, _D), jnp.float32),
    )(x, a0, a1, deg2, w_self, wn0, wn1, b2)


def kernel(in_feat, edge_index, W_self, W_neigh, b):
    xp = jnp.pad(in_feat, ((0, _NPAD - _N), (0, 0)))
    x_lo = xp[:, :_HALF]
    x_hi = xp[:, _HALF:]
    src4 = edge_index[0].reshape(_TILES, _NB, 1, _K)
    dst4 = edge_index[1].reshape(_TILES, _NB, 1, _K)
    agg_lo, agg_hi, deg = _sc_aggregate(x_lo, x_hi, src4, dst4)
    return _tc_combine(in_feat, agg_lo, agg_hi, deg, W_self, W_neigh, b)
